# Initial kernel scaffold; baseline (speedup 1.0000x reference)
import jax
import jax.numpy as jnp
from jax.experimental import pallas as pl


def kernel(mask_logits, sem_seg_logits, gt_boxes, gt_classes):
    raise NotImplementedError("write your pallas kernel here")


# speedup vs baseline: 1.4065x; 1.4065x over previous
import functools

import jax
import jax.numpy as jnp
import numpy as np
from jax.experimental import pallas as pl
from jax.experimental.pallas import tpu as pltpu

_N = 50
_M = 100
_H = 512
_W = 512
_SEM = 133
_THING = 80
_STUFF = _SEM - _THING
_CH = _STUFF + _N
_WIN = 128
_EPS = 1000.0 * float(np.finfo(np.float32).eps)


def _resize_weights(out_pos, k, box_len):
    inv = jnp.float32(_M) / box_len
    kernel_scale = jnp.maximum(inv, 1.0)
    sample = (out_pos + 0.5) * inv - 0.5
    x = jnp.abs(sample - k) / kernel_scale
    w = jnp.maximum(0.0, 1.0 - x)
    total = jnp.sum(w, axis=0, keepdims=True)
    w = jnp.where(
        jnp.abs(total) > _EPS,
        w / jnp.where(total != 0.0, total, 1.0),
        0.0,
    )
    return w


def _body(smap_ref, boxes_ref, sem_ref, mask_ref, out_ref):
    j = pl.program_id(0)

    @pl.when(j < _STUFF)
    def _copy_stuff():
        out_ref[...] = sem_ref[...]

    @pl.when(j >= _STUFF)
    def _thing_channel():
        x0 = boxes_ref[j, 0]
        y0 = boxes_ref[j, 1]
        x1 = boxes_ref[j, 2]
        y1 = boxes_ref[j, 3]
        bw = (x1 - x0 + 1).astype(jnp.float32)
        bh = (y1 - y0 + 1).astype(jnp.float32)
        row_start = jnp.minimum(y0 // 8, (_H - _WIN) // 8) * 8

        ky = jax.lax.broadcasted_iota(jnp.int32, (_M, _WIN), 0).astype(jnp.float32)
        jy = jax.lax.broadcasted_iota(jnp.int32, (_M, _WIN), 1) + row_start
        wy = _resize_weights((jy - y0).astype(jnp.float32), ky, bh)
        wy = jnp.where((jy >= y0) & (jy <= y1), wy, 0.0)

        kx = jax.lax.broadcasted_iota(jnp.int32, (_M, _W), 0).astype(jnp.float32)
        jx = jax.lax.broadcasted_iota(jnp.int32, (_M, _W), 1)
        wx = _resize_weights((jx - x0).astype(jnp.float32), kx, bw)
        wx = jnp.where((jx >= x0) & (jx <= x1), wx, 0.0)

        f = mask_ref[0, 0, :, :]
        ty = jax.lax.dot_general(
            wy, f, (((0,), (0,)), ((), ())),
            precision=jax.lax.Precision.HIGHEST,
            preferred_element_type=jnp.float32,
        )
        res = jax.lax.dot_general(
            ty, wx, (((1,), (0,)), ((), ())),
            precision=jax.lax.Precision.HIGHEST,
            preferred_element_type=jnp.float32,
        )

        sem_win = sem_ref[0, 0, pl.ds(row_start, _WIN), :]
        rows = jax.lax.broadcasted_iota(jnp.int32, (_WIN, _W), 0) + row_start
        cols = jax.lax.broadcasted_iota(jnp.int32, (_WIN, _W), 1)
        inbox = (rows >= y0) & (rows <= y1) & (cols >= x0) & (cols <= x1)
        strip = res + jnp.where(inbox, sem_win, 0.0)

        out_ref[...] = jnp.zeros((1, 1, _H, _W), jnp.float32)
        out_ref[0, 0, pl.ds(row_start, _WIN), :] = strip


def _sem_index(j, smap_ref, boxes_ref):
    return (0, smap_ref[j], 0, 0)


def _mask_index(j, smap_ref, boxes_ref):
    inst = jnp.maximum(j - _STUFF, 0)
    cls = jnp.maximum(smap_ref[j] - _STUFF, 0)
    return (inst, cls, 0, 0)


def _out_index(j, smap_ref, boxes_ref):
    return (0, j, 0, 0)


@functools.partial(jax.jit, static_argnames=())
def kernel(mask_logits, sem_seg_logits, gt_boxes, gt_classes):
    classes = gt_classes.astype(jnp.int32)
    boxes = gt_boxes.astype(jnp.int32)
    smap = jnp.concatenate(
        [jnp.arange(_STUFF, dtype=jnp.int32), classes + _STUFF])
    boxes_all = jnp.concatenate(
        [jnp.zeros((_STUFF, 4), jnp.int32), boxes], axis=0)

    grid_spec = pltpu.PrefetchScalarGridSpec(
        num_scalar_prefetch=2,
        grid=(_CH,),
        in_specs=[
            pl.BlockSpec((1, 1, _H, _W), _sem_index),
            pl.BlockSpec((1, 1, _M, _M), _mask_index),
        ],
        out_specs=pl.BlockSpec((1, 1, _H, _W), _out_index),
    )
    out = pl.pallas_call(
        _body,
        grid_spec=grid_spec,
        out_shape=jax.ShapeDtypeStruct((1, _CH, _H, _W), jnp.float32),
    )(smap, boxes_all, sem_seg_logits, mask_logits)
    return out
